# baseline (speedup 1.0000x reference)
import jax
import jax.numpy as jnp
from jax import lax
from jax.experimental import pallas as pl
from jax.experimental.pallas import tpu as pltpu
from jax.experimental.pallas import tpu_sc as plsc

_B = 16
_N = 65536
_I = 16384
_CW = 128
_NW = 32
_CHUNKS = _I // 2 // _CW


def _gather_body(t_in_hbm, idx_hbm, out_hbm, idx_v, rows_v, sem):
    c = lax.axis_index("c")
    s = lax.axis_index("s")
    wid = s * 2 + c
    b = wid // 2
    h = wid % 2
    pltpu.sync_copy(idx_hbm.at[b, pl.ds(h * _CHUNKS, _CHUNKS)], idx_v)

    def fire(j, carry):
        pltpu.async_copy(t_in_hbm.at[b].at[idx_v.at[j]], rows_v.at[j], sem)
        return carry

    lax.fori_loop(0, _CHUNKS, fire, 0)

    def drain(j, carry):
        pltpu.make_async_copy(t_in_hbm.at[b].at[idx_v.at[j]], rows_v.at[j], sem).wait()
        return carry

    lax.fori_loop(0, _CHUNKS, drain, 0)

    pltpu.sync_copy(rows_v, out_hbm.at[b, pl.ds(h * _CHUNKS, _CHUNKS)])


def kernel(t_in, t_idx):
    idx = t_idx.astype(jnp.int32).reshape(_B, _I // _CW, _CW)
    mesh = plsc.VectorSubcoreMesh(core_axis_name="c", subcore_axis_name="s")
    k = pl.kernel(
        _gather_body,
        out_type=jax.ShapeDtypeStruct((_B, _I // _CW, _CW, 3), jnp.float32),
        mesh=mesh,
        scratch_types=[
            pltpu.VMEM((_CHUNKS, _CW), jnp.int32),
            pltpu.VMEM((_CHUNKS, _CW, 3), jnp.float32),
            pltpu.SemaphoreType.DMA,
        ],
        compiler_params=pltpu.CompilerParams(use_tc_tiling_on_sc=False),
    )
    out = k(t_in, idx)
    return out.reshape(_B, _I, 3)


# speedup vs baseline: 103.3211x; 103.3211x over previous
import jax
import jax.numpy as jnp
from jax import lax
from jax.experimental import pallas as pl
from jax.experimental.pallas import tpu as pltpu
from jax.experimental.pallas import tpu_sc as plsc


def _gather_body(t5, idx5, out5, plane_v, idx_v, out_v):
    c = lax.axis_index("c")
    s = lax.axis_index("s")
    wid = s * 2 + c
    heavy = wid < 16
    b = lax.select(heavy, wid, wid - 16)
    bt = b // 8
    rb = b % 8
    pltpu.sync_copy(idx5.at[bt, :, rb, :], idx_v)

    def do_plane(p):
        pltpu.sync_copy(t5.at[p, bt, :, rb, :], plane_v)

        def row(r, carry):
            def col(j, carry2):
                n = idx_v[r, pl.ds(j * 16, 16)]
                hi = lax.shift_right_logical(n, 7)
                lo = lax.bitwise_and(n, 127)
                out_v[r, pl.ds(j * 16, 16)] = plsc.load_gather(plane_v, [hi, lo])
                return carry2

            return lax.fori_loop(0, 8, col, carry)

        lax.fori_loop(0, 128, row, 0)
        pltpu.sync_copy(out_v, out5.at[p, bt, :, rb, :])

    do_plane(lax.select(heavy, 0, 1))

    @pl.when(heavy)
    def _():
        do_plane(2)


def kernel(t_in, t_idx):
    t5 = t_in.transpose(2, 0, 1).reshape(3, 2, 8, 512, 128).transpose(0, 1, 3, 2, 4)
    idx5 = t_idx.astype(jnp.int32).reshape(2, 8, 128, 128).transpose(0, 2, 1, 3)
    mesh = plsc.VectorSubcoreMesh(core_axis_name="c", subcore_axis_name="s")
    k = pl.kernel(
        _gather_body,
        out_type=jax.ShapeDtypeStruct((3, 2, 128, 8, 128), jnp.float32),
        mesh=mesh,
        scratch_types=[
            pltpu.VMEM((512, 128), jnp.float32),
            pltpu.VMEM((128, 128), jnp.int32),
            pltpu.VMEM((128, 128), jnp.float32),
        ],
        compiler_params=pltpu.CompilerParams(
            use_tc_tiling_on_sc=False, needs_layout_passes=False
        ),
    )
    out5 = k(t5, idx5)
    return out5.transpose(1, 3, 2, 4, 0).reshape(16, 16384, 3)


# speedup vs baseline: 123.2835x; 1.1932x over previous
import jax
import jax.numpy as jnp
from jax import lax
from jax.experimental import pallas as pl
from jax.experimental.pallas import tpu as pltpu
from jax.experimental.pallas import tpu_sc as plsc


def _gather_body(t5, idx5, out5, plane_v, idx_v, out_v):
    c = lax.axis_index("c")
    s = lax.axis_index("s")
    wid = s * 2 + c
    heavy = wid < 16
    b = lax.select(heavy, wid, wid - 16)
    bt = b // 8
    rb = b % 8
    pltpu.sync_copy(idx5.at[bt, :, rb, :], idx_v)

    def do_plane(p):
        pltpu.sync_copy(t5.at[p, bt, :, rb, :], plane_v)

        @plsc.parallel_loop(0, 1024, step=1, unroll=8)
        def _(k):
            r = lax.shift_right_logical(k, 3)
            o = lax.bitwise_and(k, 7) * 16
            n = idx_v[r, pl.ds(o, 16)]
            hi = lax.shift_right_logical(n, 7)
            lo = lax.bitwise_and(n, 127)
            out_v[r, pl.ds(o, 16)] = plsc.load_gather(plane_v, [hi, lo])

        pltpu.sync_copy(out_v, out5.at[p, bt, :, rb, :])

    do_plane(lax.select(heavy, 0, 1))

    @pl.when(heavy)
    def _():
        do_plane(2)


def kernel(t_in, t_idx):
    t5 = t_in.transpose(2, 0, 1).reshape(3, 2, 8, 512, 128).transpose(0, 1, 3, 2, 4)
    idx5 = t_idx.astype(jnp.int32).reshape(2, 8, 128, 128).transpose(0, 2, 1, 3)
    mesh = plsc.VectorSubcoreMesh(core_axis_name="c", subcore_axis_name="s")
    k = pl.kernel(
        _gather_body,
        out_type=jax.ShapeDtypeStruct((3, 2, 128, 8, 128), jnp.float32),
        mesh=mesh,
        scratch_types=[
            pltpu.VMEM((512, 128), jnp.float32),
            pltpu.VMEM((128, 128), jnp.int32),
            pltpu.VMEM((128, 128), jnp.float32),
        ],
        compiler_params=pltpu.CompilerParams(
            use_tc_tiling_on_sc=False, needs_layout_passes=False
        ),
    )
    out5 = k(t5, idx5)
    return out5.transpose(1, 3, 2, 4, 0).reshape(16, 16384, 3)
